# baseline (speedup 1.0000x reference)
import functools

import jax
import jax.numpy as jnp
from jax import lax
from jax.experimental import pallas as pl
from jax.experimental.pallas import tpu as pltpu
from jax.experimental.pallas import tpu_sc as plsc

BATCH = 16384
HIST = 50
HIST_PAD = 56
EMBED_DIM = 64
PADDED_DIM = 128

NUM_WORKERS = 32
PER_WORKER = BATCH // NUM_WORKERS
GROUP = 2
NCHUNK = PER_WORKER // GROUP
NBUF = 4

_mesh = plsc.VectorSubcoreMesh(core_axis_name="c", subcore_axis_name="s")


@functools.partial(
    pl.kernel,
    mesh=_mesh,
    out_type=jax.ShapeDtypeStruct(
        (BATCH // GROUP, GROUP * HIST_PAD, PADDED_DIM), jnp.float32
    ),
    scratch_types=[pltpu.VMEM((NCHUNK, GROUP * HIST_PAD), jnp.int32)]
    + [pltpu.VMEM((GROUP * HIST_PAD, PADDED_DIM), jnp.float32) for _ in range(NBUF)]
    + [pltpu.SemaphoreType.DMA for _ in range(NBUF)]
    + [pltpu.SemaphoreType.DMA for _ in range(NBUF)],
    compiler_params=pltpu.CompilerParams(use_tc_tiling_on_sc=False),
)
def _embed_gather(idx_hbm, table_hbm, out_hbm, idx_v, *bufs_and_sems):
    bufs = bufs_and_sems[:NBUF]
    gsems = bufs_and_sems[NBUF : 2 * NBUF]
    osems = bufs_and_sems[2 * NBUF :]
    wid = lax.axis_index("s") * 2 + lax.axis_index("c")
    base = wid * PER_WORKER
    half = NBUF // 2

    pltpu.sync_copy(idx_hbm.at[pl.ds(base // GROUP, NCHUNK)], idx_v)

    for b in range(NBUF):
        pltpu.async_copy(table_hbm.at[idx_v.at[b]], bufs[b], gsems[b])

    def body(j0, carry):
        for b in range(NBUF):
            j = j0 * NBUF + b
            pltpu.make_async_copy(
                table_hbm.at[idx_v.at[0]], bufs[b], gsems[b]
            ).wait()
            pltpu.async_copy(bufs[b], out_hbm.at[base // GROUP + j], osems[b])
            jn = j + half
            bn = (b + half) % NBUF

            @pl.when(jnp.logical_and(j >= half, jn < NCHUNK))
            def _():
                pltpu.make_async_copy(
                    bufs[bn], out_hbm.at[base // GROUP], osems[bn]
                ).wait()
                pltpu.async_copy(table_hbm.at[idx_v.at[jn]], bufs[bn], gsems[bn])

        return carry

    lax.fori_loop(0, NCHUNK // NBUF, body, 0)

    for b in range(NBUF):
        pltpu.make_async_copy(bufs[b], out_hbm.at[base // GROUP], osems[b]).wait()


def kernel(item_inputs, item_embedding):
    fill = (
        jnp.arange(BATCH, dtype=jnp.int32)[:, None] * 61
        + jnp.arange(HIST_PAD - HIST, dtype=jnp.int32)[None, :]
    ) % 1000000
    idx = jnp.concatenate([item_inputs.astype(jnp.int32), fill], axis=1)
    tab = jnp.pad(item_embedding, ((0, 0), (0, PADDED_DIM - EMBED_DIM)))
    idx = idx.reshape(BATCH // GROUP, GROUP * HIST_PAD)
    out = _embed_gather(idx, tab)
    out = out.reshape(BATCH, HIST_PAD, PADDED_DIM)
    return lax.slice(out, (0, 0, 0), (BATCH, HIST, EMBED_DIM))


# speedup vs baseline: 1.0015x; 1.0015x over previous
import functools

import jax
import jax.numpy as jnp
from jax import lax
from jax.experimental import pallas as pl
from jax.experimental.pallas import tpu as pltpu
from jax.experimental.pallas import tpu_sc as plsc

BATCH = 16384
HIST = 50
HIST_PAD = 56
EMBED_DIM = 64
PADDED_DIM = 128

NUM_WORKERS = 32
PER_WORKER = BATCH // NUM_WORKERS
NBUF = 8

_mesh = plsc.VectorSubcoreMesh(core_axis_name="c", subcore_axis_name="s")


@functools.partial(
    pl.kernel,
    mesh=_mesh,
    out_type=jax.ShapeDtypeStruct((BATCH, HIST_PAD, PADDED_DIM), jnp.float32),
    scratch_types=[pltpu.VMEM((PER_WORKER, HIST_PAD), jnp.int32)]
    + [pltpu.VMEM((HIST_PAD, PADDED_DIM), jnp.float32) for _ in range(NBUF)]
    + [pltpu.SemaphoreType.DMA for _ in range(NBUF)]
    + [pltpu.SemaphoreType.DMA for _ in range(NBUF)],
    compiler_params=pltpu.CompilerParams(use_tc_tiling_on_sc=False),
)
def _embed_gather(idx_hbm, table_hbm, out_hbm, idx_v, *bufs_and_sems):
    bufs = bufs_and_sems[:NBUF]
    gsems = bufs_and_sems[NBUF : 2 * NBUF]
    osems = bufs_and_sems[2 * NBUF :]
    wid = lax.axis_index("s") * 2 + lax.axis_index("c")
    base = wid * PER_WORKER
    half = NBUF // 2

    pltpu.sync_copy(idx_hbm.at[pl.ds(base, PER_WORKER)], idx_v)

    for b in range(NBUF):
        pltpu.async_copy(table_hbm.at[idx_v.at[b]], bufs[b], gsems[b])

    def body(j0, carry):
        for b in range(NBUF):
            j = j0 * NBUF + b
            pltpu.make_async_copy(
                table_hbm.at[idx_v.at[0]], bufs[b], gsems[b]
            ).wait()
            pltpu.async_copy(bufs[b], out_hbm.at[base + j], osems[b])
            jn = j + half
            bn = (b + half) % NBUF

            @pl.when(jnp.logical_and(j >= half, jn < PER_WORKER))
            def _():
                pltpu.make_async_copy(
                    bufs[bn], out_hbm.at[base], osems[bn]
                ).wait()
                pltpu.async_copy(table_hbm.at[idx_v.at[jn]], bufs[bn], gsems[bn])

        return carry

    lax.fori_loop(0, PER_WORKER // NBUF, body, 0)

    for b in range(NBUF):
        pltpu.make_async_copy(bufs[b], out_hbm.at[base], osems[b]).wait()


def kernel(item_inputs, item_embedding):
    fill = (
        jnp.arange(BATCH, dtype=jnp.int32)[:, None] * 61
        + jnp.arange(HIST_PAD - HIST, dtype=jnp.int32)[None, :]
    ) % 1000000
    idx = jnp.concatenate([item_inputs.astype(jnp.int32), fill], axis=1)
    tab = jnp.pad(item_embedding, ((0, 0), (0, PADDED_DIM - EMBED_DIM)))
    out = _embed_gather(idx, tab)
    return lax.slice(out, (0, 0, 0), (BATCH, HIST, EMBED_DIM))
